# baseline (speedup 1.0000x reference)
import functools

import jax
import jax.numpy as jnp
from jax import lax
from jax.experimental import pallas as pl
from jax.experimental.pallas import tpu as pltpu
from jax.experimental.pallas import tpu_sc as plsc

N_NODES = 10000
N_PAD = 10240
F_IN = 128
F_HID = 256
F_OUT = 128
N_EDGES = 320000
NUM_TILES = 32
CHUNK = 128
C_CHUNKS = 80
PCHUNK = 128
PC_CHUNKS = 80
PSLAB = 16
NBUF = 2
E_PAD = NUM_TILES * CHUNK * C_CHUNKS
ROWS_PER_TILE = N_PAD // 16
N_JUNK = N_PAD - N_NODES

_MESH = dict(core_axis_name="c", subcore_axis_name="s")


def _sc_degree(dst_t, ones_rows, zeros_feat):
    @functools.partial(
        pl.kernel,
        out_type=jax.ShapeDtypeStruct((2, N_PAD), jnp.float32),
        mesh=plsc.VectorSubcoreMesh(**_MESH),
        scratch_types=[
            pltpu.VMEM_SHARED((N_PAD,), jnp.float32),
            pltpu.VMEM((C_CHUNKS, CHUNK), jnp.int32),
            pltpu.VMEM((CHUNK,), jnp.float32),
            pltpu.SemaphoreType.DMA,
        ],
    )
    def k(dst_hbm, ones_hbm, zeros_hbm, out_hbm, acc_sh, didx, ones_v, sem_s):
        ci = lax.axis_index("c")
        si = lax.axis_index("s")
        wid = si * 2 + ci
        base = si * ROWS_PER_TILE
        pltpu.sync_copy(zeros_hbm.at[pl.ds(base, ROWS_PER_TILE)],
                        acc_sh.at[pl.ds(base, ROWS_PER_TILE)])
        pltpu.sync_copy(dst_hbm.at[wid], didx)
        pltpu.sync_copy(ones_hbm, ones_v)
        plsc.subcore_barrier()

        def fire(j, carry):
            pltpu.async_copy(ones_v, acc_sh.at[didx.at[j]], sem_s, add=True)
            return carry

        lax.fori_loop(0, C_CHUNKS, fire, 0)

        def drain(j, carry):
            pltpu.make_async_copy(ones_v, acc_sh.at[didx.at[j]], sem_s).wait()
            return carry

        lax.fori_loop(0, C_CHUNKS, drain, 0)

        plsc.subcore_barrier()
        pltpu.sync_copy(acc_sh.at[pl.ds(base, ROWS_PER_TILE)],
                        out_hbm.at[ci, pl.ds(base, ROWS_PER_TILE)])

    return k(dst_t, ones_rows, zeros_feat)


def _sc_propagate(table, src_t, dst_t, zeros_feat):
    @functools.partial(
        pl.kernel,
        out_type=jax.ShapeDtypeStruct((2, N_PAD, F_IN), jnp.float32),
        mesh=plsc.VectorSubcoreMesh(**_MESH),
        scratch_types=[
            pltpu.VMEM_SHARED((N_PAD, F_IN), jnp.float32),
            pltpu.VMEM((2, PSLAB, PCHUNK), jnp.int32),
            pltpu.VMEM((2, PSLAB, PCHUNK), jnp.int32),
            pltpu.VMEM((NBUF, PCHUNK, F_IN), jnp.float32),
        ] + [pltpu.SemaphoreType.DMA] * (2 * NBUF + 2),
    )
    def k(tab_hbm, src_hbm, dst_hbm, zeros_hbm, out_hbm,
          acc_sh, sidx, didx, rows, *sems):
        sem_g = sems[:NBUF]
        sem_s = sems[NBUF:2 * NBUF]
        sem_z, sem_i = sems[2 * NBUF:]
        ci = lax.axis_index("c")
        si = lax.axis_index("s")
        wid = si * 2 + ci
        base = si * ROWS_PER_TILE
        n_pass = PC_CHUNKS // PSLAB
        pltpu.async_copy(zeros_hbm.at[pl.ds(base, ROWS_PER_TILE)],
                         acc_sh.at[pl.ds(base, ROWS_PER_TILE)], sem_z)
        pltpu.sync_copy(src_hbm.at[wid, pl.ds(0, PSLAB)], sidx.at[0])
        pltpu.sync_copy(dst_hbm.at[wid, pl.ds(0, PSLAB)], didx.at[0])
        for b in range(NBUF):
            pltpu.async_copy(tab_hbm.at[sidx.at[0, b]], rows.at[b], sem_g[b])
        pltpu.make_async_copy(zeros_hbm.at[pl.ds(base, ROWS_PER_TILE)],
                              acc_sh.at[pl.ds(base, ROWS_PER_TILE)],
                              sem_z).wait()
        plsc.subcore_barrier()

        for p in range(n_pass):
            bk = p % 2
            sx = sidx.at[bk]
            dx = didx.at[bk]
            if p + 1 < n_pass:
                pltpu.async_copy(src_hbm.at[wid, pl.ds((p + 1) * PSLAB, PSLAB)],
                                 sidx.at[1 - bk], sem_i)
                pltpu.async_copy(dst_hbm.at[wid, pl.ds((p + 1) * PSLAB, PSLAB)],
                                 didx.at[1 - bk], sem_i)

            def step(i, carry, sx=sx, dx=dx):
                jb = NBUF * i
                for b in range(NBUF):
                    pltpu.make_async_copy(tab_hbm.at[sx.at[jb + b]],
                                          rows.at[b], sem_g[b]).wait()
                    pltpu.async_copy(rows.at[b], acc_sh.at[dx.at[jb + b]],
                                     sem_s[b], add=True)
                for b in range(NBUF):
                    pltpu.make_async_copy(rows.at[b], acc_sh.at[dx.at[jb + b]],
                                          sem_s[b]).wait()
                    pltpu.async_copy(tab_hbm.at[sx.at[jb + b + NBUF]],
                                     rows.at[b], sem_g[b])
                return carry

            lax.fori_loop(0, PSLAB // NBUF - 1, step, 0)

            jb = PSLAB - NBUF
            for b in range(NBUF):
                pltpu.make_async_copy(tab_hbm.at[sx.at[jb + b]],
                                      rows.at[b], sem_g[b]).wait()
                pltpu.async_copy(rows.at[b], acc_sh.at[dx.at[jb + b]],
                                 sem_s[b], add=True)
            for b in range(NBUF):
                pltpu.make_async_copy(rows.at[b], acc_sh.at[dx.at[jb + b]],
                                      sem_s[b]).wait()
                if p + 1 < n_pass:
                    if b == 0:
                        pltpu.make_async_copy(
                            src_hbm.at[wid, pl.ds((p + 1) * PSLAB, PSLAB)],
                            sidx.at[1 - bk], sem_i).wait()
                        pltpu.make_async_copy(
                            dst_hbm.at[wid, pl.ds((p + 1) * PSLAB, PSLAB)],
                            didx.at[1 - bk], sem_i).wait()
                    pltpu.async_copy(tab_hbm.at[sidx.at[1 - bk, b]],
                                     rows.at[b], sem_g[b])

        plsc.subcore_barrier()
        pltpu.sync_copy(acc_sh.at[pl.ds(base, ROWS_PER_TILE)],
                        out_hbm.at[ci, pl.ds(base, ROWS_PER_TILE)])

    return k(table, src_t, dst_t, zeros_feat)


def _dinv_of(cnt_blk):
    return lax.rsqrt(1.0 + cnt_blk[0, :] + cnt_blk[1, :])


def _tc_prescale(cnt, x_pad):
    R = 1024

    def body(cnt_ref, x_ref, y_ref):
        dinv = _dinv_of(cnt_ref[...])
        y_ref[...] = x_ref[...] * dinv[:, None]

    return pl.pallas_call(
        body,
        grid=(N_PAD // R,),
        in_specs=[
            pl.BlockSpec((2, R), lambda i: (0, i)),
            pl.BlockSpec((R, F_IN), lambda i: (i, 0)),
        ],
        out_specs=pl.BlockSpec((R, F_IN), lambda i: (i, 0)),
        out_shape=jax.ShapeDtypeStruct((N_PAD, F_IN), jnp.float32),
    )(cnt, x_pad)


def _tc_mlp(cnt, y1, acc1, W1, b1, W2):
    R = 512

    def body(cnt_ref, y_ref, acc_ref, w1_ref, b1_ref, w2_ref, z_ref):
        dinv = _dinv_of(cnt_ref[...])
        p = (acc_ref[0] + acc_ref[1] + y_ref[...]) * dinv[:, None]
        h = jnp.dot(p, w1_ref[...], preferred_element_type=jnp.float32)
        h = jnp.maximum(h + b1_ref[...], 0.0)
        t = jnp.dot(h, w2_ref[...], preferred_element_type=jnp.float32)
        z_ref[...] = t * dinv[:, None]

    return pl.pallas_call(
        body,
        grid=(N_PAD // R,),
        in_specs=[
            pl.BlockSpec((2, R), lambda i: (0, i)),
            pl.BlockSpec((R, F_IN), lambda i: (i, 0)),
            pl.BlockSpec((2, R, F_IN), lambda i: (0, i, 0)),
            pl.BlockSpec((F_IN, F_HID), lambda i: (0, 0)),
            pl.BlockSpec((1, F_HID), lambda i: (0, 0)),
            pl.BlockSpec((F_HID, F_OUT), lambda i: (0, 0)),
        ],
        out_specs=pl.BlockSpec((R, F_IN), lambda i: (i, 0)),
        out_shape=jax.ShapeDtypeStruct((N_PAD, F_IN), jnp.float32),
    )(cnt, y1, acc1, W1, b1, W2)


def _tc_epilogue(cnt, z, acc2, b2):
    R = 1024

    def body(cnt_ref, z_ref, acc_ref, b2_ref, out_ref):
        dinv = _dinv_of(cnt_ref[...])
        out_ref[...] = (acc_ref[0] + acc_ref[1] + z_ref[...]) * dinv[:, None] + b2_ref[...]

    return pl.pallas_call(
        body,
        grid=(N_PAD // R,),
        in_specs=[
            pl.BlockSpec((2, R), lambda i: (0, i)),
            pl.BlockSpec((R, F_OUT), lambda i: (i, 0)),
            pl.BlockSpec((2, R, F_OUT), lambda i: (0, i, 0)),
            pl.BlockSpec((1, F_OUT), lambda i: (0, 0)),
        ],
        out_specs=pl.BlockSpec((R, F_OUT), lambda i: (i, 0)),
        out_shape=jax.ShapeDtypeStruct((N_PAD, F_OUT), jnp.float32),
    )(cnt, z, acc2, b2)


def kernel(x, edge_index, W1, b1, W2, b2):
    src = edge_index[0].astype(jnp.int32)
    dst = edge_index[1].astype(jnp.int32)
    pad = N_NODES + (jnp.arange(E_PAD - N_EDGES, dtype=jnp.int32) % N_JUNK)
    src_t = jnp.concatenate([src, pad]).reshape(NUM_TILES, PC_CHUNKS, PCHUNK)
    dst_p = jnp.concatenate([dst, pad])
    dst_t = dst_p.reshape(NUM_TILES, PC_CHUNKS, PCHUNK)
    dst_deg = dst_p.reshape(NUM_TILES, C_CHUNKS, CHUNK)
    x_pad = jnp.pad(x, ((0, N_PAD - N_NODES), (0, 0)))
    zeros_feat = jnp.zeros((N_PAD, F_IN), jnp.float32)
    ones_rows = jnp.ones((CHUNK,), jnp.float32)

    zeros_n = jnp.zeros((N_PAD,), jnp.float32)
    cnt = _sc_degree(dst_deg, ones_rows, zeros_n)
    y1 = _tc_prescale(cnt, x_pad)
    acc1 = _sc_propagate(y1, src_t, dst_t, zeros_feat)
    z = _tc_mlp(cnt, y1, acc1, W1, b1.reshape(1, F_HID), W2)
    acc2 = _sc_propagate(z, src_t, dst_t, zeros_feat)
    logits = _tc_epilogue(cnt, z, acc2, b2.reshape(1, F_OUT))
    return logits[:N_NODES]


# speedup vs baseline: 1.2054x; 1.2054x over previous
import functools

import jax
import jax.numpy as jnp
from jax import lax
from jax.experimental import pallas as pl
from jax.experimental.pallas import tpu as pltpu
from jax.experimental.pallas import tpu_sc as plsc

N_NODES = 10000
N_PAD = 10240
F_IN = 128
F_HID = 256
F_OUT = 128
N_EDGES = 320000
NUM_TILES = 32
CHUNK = 128
C_CHUNKS = 80
PCHUNK = 80
PC_CHUNKS = 128
PSLAB = 8
NBUF = 4
E_PAD = NUM_TILES * CHUNK * C_CHUNKS
ROWS_PER_TILE = N_PAD // 16
N_JUNK = N_PAD - N_NODES

_MESH = dict(core_axis_name="c", subcore_axis_name="s")


def _sc_degree(dst_t, ones_rows, zeros_feat):
    @functools.partial(
        pl.kernel,
        out_type=jax.ShapeDtypeStruct((2, N_PAD), jnp.float32),
        mesh=plsc.VectorSubcoreMesh(**_MESH),
        scratch_types=[
            pltpu.VMEM_SHARED((N_PAD,), jnp.float32),
            pltpu.VMEM((C_CHUNKS, CHUNK), jnp.int32),
            pltpu.VMEM((CHUNK,), jnp.float32),
            pltpu.SemaphoreType.DMA,
        ],
    )
    def k(dst_hbm, ones_hbm, zeros_hbm, out_hbm, acc_sh, didx, ones_v, sem_s):
        ci = lax.axis_index("c")
        si = lax.axis_index("s")
        wid = si * 2 + ci
        base = si * ROWS_PER_TILE
        pltpu.sync_copy(zeros_hbm.at[pl.ds(base, ROWS_PER_TILE)],
                        acc_sh.at[pl.ds(base, ROWS_PER_TILE)])
        pltpu.sync_copy(dst_hbm.at[wid], didx)
        pltpu.sync_copy(ones_hbm, ones_v)
        plsc.subcore_barrier()

        def fire(j, carry):
            pltpu.async_copy(ones_v, acc_sh.at[didx.at[j]], sem_s, add=True)
            return carry

        lax.fori_loop(0, C_CHUNKS, fire, 0)

        def drain(j, carry):
            pltpu.make_async_copy(ones_v, acc_sh.at[didx.at[j]], sem_s).wait()
            return carry

        lax.fori_loop(0, C_CHUNKS, drain, 0)

        plsc.subcore_barrier()
        pltpu.sync_copy(acc_sh.at[pl.ds(base, ROWS_PER_TILE)],
                        out_hbm.at[ci, pl.ds(base, ROWS_PER_TILE)])

    return k(dst_t, ones_rows, zeros_feat)


def _sc_propagate(table, src_t, dst_t, zeros_feat):
    @functools.partial(
        pl.kernel,
        out_type=jax.ShapeDtypeStruct((2, N_PAD, F_IN), jnp.float32),
        mesh=plsc.VectorSubcoreMesh(**_MESH),
        scratch_types=[
            pltpu.VMEM_SHARED((N_PAD, F_IN), jnp.float32),
            pltpu.VMEM((2, PSLAB, PCHUNK), jnp.int32),
            pltpu.VMEM((2, PSLAB, PCHUNK), jnp.int32),
            pltpu.VMEM((NBUF, PCHUNK, F_IN), jnp.float32),
        ] + [pltpu.SemaphoreType.DMA] * (2 * NBUF + 2),
    )
    def k(tab_hbm, src_hbm, dst_hbm, zeros_hbm, out_hbm,
          acc_sh, sidx, didx, rows, *sems):
        sem_g = sems[:NBUF]
        sem_s = sems[NBUF:2 * NBUF]
        sem_z, sem_i = sems[2 * NBUF:]
        ci = lax.axis_index("c")
        si = lax.axis_index("s")
        wid = si * 2 + ci
        base = si * ROWS_PER_TILE
        n_pass = PC_CHUNKS // PSLAB
        pltpu.async_copy(zeros_hbm.at[pl.ds(base, ROWS_PER_TILE)],
                         acc_sh.at[pl.ds(base, ROWS_PER_TILE)], sem_z)
        pltpu.sync_copy(src_hbm.at[wid, pl.ds(0, PSLAB)], sidx.at[0])
        pltpu.sync_copy(dst_hbm.at[wid, pl.ds(0, PSLAB)], didx.at[0])
        for b in range(NBUF):
            pltpu.async_copy(tab_hbm.at[sidx.at[0, b]], rows.at[b], sem_g[b])
        pltpu.make_async_copy(zeros_hbm.at[pl.ds(base, ROWS_PER_TILE)],
                              acc_sh.at[pl.ds(base, ROWS_PER_TILE)],
                              sem_z).wait()
        plsc.subcore_barrier()

        for p in range(n_pass):
            bk = p % 2
            sx = sidx.at[bk]
            dx = didx.at[bk]
            if p + 1 < n_pass:
                pltpu.async_copy(src_hbm.at[wid, pl.ds((p + 1) * PSLAB, PSLAB)],
                                 sidx.at[1 - bk], sem_i)
                pltpu.async_copy(dst_hbm.at[wid, pl.ds((p + 1) * PSLAB, PSLAB)],
                                 didx.at[1 - bk], sem_i)

            def step(i, carry, sx=sx, dx=dx):
                jb = NBUF * i
                for b in range(NBUF):
                    pltpu.make_async_copy(tab_hbm.at[sx.at[jb + b]],
                                          rows.at[b], sem_g[b]).wait()
                    pltpu.async_copy(rows.at[b], acc_sh.at[dx.at[jb + b]],
                                     sem_s[b], add=True)
                for b in range(NBUF):
                    pltpu.make_async_copy(rows.at[b], acc_sh.at[dx.at[jb + b]],
                                          sem_s[b]).wait()
                    pltpu.async_copy(tab_hbm.at[sx.at[jb + b + NBUF]],
                                     rows.at[b], sem_g[b])
                return carry

            lax.fori_loop(0, PSLAB // NBUF - 1, step, 0)

            jb = PSLAB - NBUF
            for b in range(NBUF):
                pltpu.make_async_copy(tab_hbm.at[sx.at[jb + b]],
                                      rows.at[b], sem_g[b]).wait()
                pltpu.async_copy(rows.at[b], acc_sh.at[dx.at[jb + b]],
                                 sem_s[b], add=True)
            for b in range(NBUF):
                pltpu.make_async_copy(rows.at[b], acc_sh.at[dx.at[jb + b]],
                                      sem_s[b]).wait()
                if p + 1 < n_pass:
                    if b == 0:
                        pltpu.make_async_copy(
                            src_hbm.at[wid, pl.ds((p + 1) * PSLAB, PSLAB)],
                            sidx.at[1 - bk], sem_i).wait()
                        pltpu.make_async_copy(
                            dst_hbm.at[wid, pl.ds((p + 1) * PSLAB, PSLAB)],
                            didx.at[1 - bk], sem_i).wait()
                    pltpu.async_copy(tab_hbm.at[sidx.at[1 - bk, b]],
                                     rows.at[b], sem_g[b])

        plsc.subcore_barrier()
        pltpu.sync_copy(acc_sh.at[pl.ds(base, ROWS_PER_TILE)],
                        out_hbm.at[ci, pl.ds(base, ROWS_PER_TILE)])

    return k(table, src_t, dst_t, zeros_feat)


def _dinv_of(cnt_blk):
    return lax.rsqrt(1.0 + cnt_blk[0, :] + cnt_blk[1, :])


def _tc_prescale(cnt, x_pad):
    R = 1024

    def body(cnt_ref, x_ref, y_ref):
        dinv = _dinv_of(cnt_ref[...])
        y_ref[...] = x_ref[...] * dinv[:, None]

    return pl.pallas_call(
        body,
        grid=(N_PAD // R,),
        in_specs=[
            pl.BlockSpec((2, R), lambda i: (0, i)),
            pl.BlockSpec((R, F_IN), lambda i: (i, 0)),
        ],
        out_specs=pl.BlockSpec((R, F_IN), lambda i: (i, 0)),
        out_shape=jax.ShapeDtypeStruct((N_PAD, F_IN), jnp.float32),
    )(cnt, x_pad)


def _tc_mlp(cnt, y1, acc1, W1, b1, W2):
    R = 512

    def body(cnt_ref, y_ref, acc_ref, w1_ref, b1_ref, w2_ref, z_ref):
        dinv = _dinv_of(cnt_ref[...])
        p = (acc_ref[0] + acc_ref[1] + y_ref[...]) * dinv[:, None]
        h = jnp.dot(p, w1_ref[...], preferred_element_type=jnp.float32)
        h = jnp.maximum(h + b1_ref[...], 0.0)
        t = jnp.dot(h, w2_ref[...], preferred_element_type=jnp.float32)
        z_ref[...] = t * dinv[:, None]

    return pl.pallas_call(
        body,
        grid=(N_PAD // R,),
        in_specs=[
            pl.BlockSpec((2, R), lambda i: (0, i)),
            pl.BlockSpec((R, F_IN), lambda i: (i, 0)),
            pl.BlockSpec((2, R, F_IN), lambda i: (0, i, 0)),
            pl.BlockSpec((F_IN, F_HID), lambda i: (0, 0)),
            pl.BlockSpec((1, F_HID), lambda i: (0, 0)),
            pl.BlockSpec((F_HID, F_OUT), lambda i: (0, 0)),
        ],
        out_specs=pl.BlockSpec((R, F_IN), lambda i: (i, 0)),
        out_shape=jax.ShapeDtypeStruct((N_PAD, F_IN), jnp.float32),
    )(cnt, y1, acc1, W1, b1, W2)


def _tc_epilogue(cnt, z, acc2, b2):
    R = 1024

    def body(cnt_ref, z_ref, acc_ref, b2_ref, out_ref):
        dinv = _dinv_of(cnt_ref[...])
        out_ref[...] = (acc_ref[0] + acc_ref[1] + z_ref[...]) * dinv[:, None] + b2_ref[...]

    return pl.pallas_call(
        body,
        grid=(N_PAD // R,),
        in_specs=[
            pl.BlockSpec((2, R), lambda i: (0, i)),
            pl.BlockSpec((R, F_OUT), lambda i: (i, 0)),
            pl.BlockSpec((2, R, F_OUT), lambda i: (0, i, 0)),
            pl.BlockSpec((1, F_OUT), lambda i: (0, 0)),
        ],
        out_specs=pl.BlockSpec((R, F_OUT), lambda i: (i, 0)),
        out_shape=jax.ShapeDtypeStruct((N_PAD, F_OUT), jnp.float32),
    )(cnt, z, acc2, b2)


def kernel(x, edge_index, W1, b1, W2, b2):
    src = edge_index[0].astype(jnp.int32)
    dst = edge_index[1].astype(jnp.int32)
    pad = N_NODES + (jnp.arange(E_PAD - N_EDGES, dtype=jnp.int32) % N_JUNK)
    src_t = jnp.concatenate([src, pad]).reshape(NUM_TILES, PC_CHUNKS, PCHUNK)
    dst_p = jnp.concatenate([dst, pad])
    dst_t = dst_p.reshape(NUM_TILES, PC_CHUNKS, PCHUNK)
    dst_deg = dst_p.reshape(NUM_TILES, C_CHUNKS, CHUNK)
    x_pad = jnp.pad(x, ((0, N_PAD - N_NODES), (0, 0)))
    zeros_feat = jnp.zeros((N_PAD, F_IN), jnp.float32)
    ones_rows = jnp.ones((CHUNK,), jnp.float32)

    zeros_n = jnp.zeros((N_PAD,), jnp.float32)
    cnt = _sc_degree(dst_deg, ones_rows, zeros_n)
    y1 = _tc_prescale(cnt, x_pad)
    acc1 = _sc_propagate(y1, src_t, dst_t, zeros_feat)
    z = _tc_mlp(cnt, y1, acc1, W1, b1.reshape(1, F_HID), W2)
    acc2 = _sc_propagate(z, src_t, dst_t, zeros_feat)
    logits = _tc_epilogue(cnt, z, acc2, b2.reshape(1, F_OUT))
    return logits[:N_NODES]
